# Initial kernel scaffold; baseline (speedup 1.0000x reference)
import jax
import jax.numpy as jnp
from jax.experimental import pallas as pl


def kernel(x, edge_src_local, edge_w, bias):
    raise NotImplementedError("write your pallas kernel here")


# speedup vs baseline: 2.2595x; 2.2595x over previous
import functools

import jax
import jax.numpy as jnp
from jax import lax
from jax.experimental import pallas as pl
from jax.experimental.pallas import tpu as pltpu
from jax.experimental.pallas import tpu_sc as plsc

NL = 8
N = 4096
K = 16
B = 256
NC, NS, LANES = 2, 16, 16
NW = NC * NS
NPW = N // NW
CH = 8
NCHUNK = NPW // CH
ROWS = CH * K
TB = B // LANES


def _layer_body(apply_act, vals_hbm, src_hbm, w_hbm, bias_hbm, out_hbm,
                idx_v, w_v, bias_v, rows_v, acc_v, sem):
    wid = lax.axis_index("s") * NC + lax.axis_index("c")
    base = wid * NPW
    ebase = base * K

    pltpu.sync_copy(src_hbm.at[pl.ds(ebase, NPW * K)], idx_v)
    pltpu.sync_copy(w_hbm.at[pl.ds(ebase, NPW * K)], w_v)
    pltpu.sync_copy(bias_hbm.at[pl.ds(base, NPW)], bias_v)

    def chunk_body(c, carry):
        pltpu.async_copy(
            vals_hbm.at[idx_v.at[pl.ds(c * ROWS, ROWS)]], rows_v, sem
        ).wait()

        def neuron_body(j, carry2):
            e0 = c * ROWS + j * K
            wvs = [
                plsc.load_gather(
                    w_v, [jnp.full((LANES,), e0 + k, dtype=jnp.int32)])
                for k in range(K)
            ]
            bv = plsc.load_gather(
                bias_v, [jnp.full((LANES,), c * CH + j, dtype=jnp.int32)])

            def t_body(t, carry3):
                acc = bv
                for k in range(K):
                    acc = acc + wvs[k] * rows_v[j * K + k,
                                                pl.ds(t * LANES, LANES)]
                if apply_act:
                    acc = acc / (1.0 + jnp.exp(-acc))
                acc_v[j, pl.ds(t * LANES, LANES)] = acc
                return carry3

            return lax.fori_loop(0, TB, t_body, carry2)

        lax.fori_loop(0, CH, neuron_body, 0)
        pltpu.sync_copy(acc_v, out_hbm.at[pl.ds(base + c * CH, CH)])
        return carry

    lax.fori_loop(0, NCHUNK, chunk_body, 0)


def _make_layer(apply_act, interpret=False):
    return pl.kernel(
        functools.partial(_layer_body, apply_act),
        out_type=jax.ShapeDtypeStruct((N, B), jnp.float32),
        mesh=plsc.VectorSubcoreMesh(
            core_axis_name="c", subcore_axis_name="s",
            num_cores=NC, num_subcores=NS),
        scratch_types=[
            pltpu.VMEM((NPW * K,), jnp.int32),
            pltpu.VMEM((NPW * K,), jnp.float32),
            pltpu.VMEM((NPW,), jnp.float32),
            pltpu.VMEM((ROWS, B), jnp.float32),
            pltpu.VMEM((CH, B), jnp.float32),
            pltpu.SemaphoreType.DMA,
        ],
        compiler_params=pltpu.CompilerParams(needs_layout_passes=False),
        interpret=interpret,
    )


@jax.jit
def kernel(x, edge_src_local, edge_w, bias):
    vals = x.T
    hidden = _make_layer(True)
    final = _make_layer(False)
    for l in range(NL - 1):
        fn = hidden if l < NL - 2 else final
        vals = fn(vals, edge_src_local[l].reshape(-1),
                  edge_w[l].reshape(-1), bias[l])
    return vals.T


# speedup vs baseline: 5.0270x; 2.2248x over previous
import functools

import jax
import jax.numpy as jnp
from jax import lax
from jax.experimental import pallas as pl
from jax.experimental.pallas import tpu as pltpu
from jax.experimental.pallas import tpu_sc as plsc

NL = 8
N = 4096
K = 16
B = 256
NC, NS, LANES = 2, 16, 16
NW = NC * NS
NPW = N // NW
CH = 8
NCHUNK = NPW // CH
ROWS = CH * K
TB = B // LANES


def _layer_body(apply_act, vals_hbm, src_hbm, w_hbm, bias_hbm, out_hbm,
                idx_v, w_v, bias_v, rows0, rows1, acc0, acc1,
                gsem0, gsem1, ssem):
    wid = lax.axis_index("s") * NC + lax.axis_index("c")
    base = wid * NPW
    ebase = base * K

    pltpu.sync_copy(src_hbm.at[pl.ds(ebase, NPW * K)], idx_v)
    pltpu.sync_copy(w_hbm.at[pl.ds(ebase, NPW * K)], w_v)
    pltpu.sync_copy(bias_hbm.at[pl.ds(base, NPW)], bias_v)

    rows = (rows0, rows1)
    acc = (acc0, acc1)
    gsem = (gsem0, gsem1)

    def gather(c):
        return pltpu.async_copy(
            vals_hbm.at[idx_v.at[pl.ds(c * ROWS, ROWS)]], rows[c % 2],
            gsem[c % 2])

    def compute(c):
        rows_v, acc_v = rows[c % 2], acc[c % 2]

        @plsc.parallel_loop(0, CH)
        def _neuron(j):
            e0 = c * ROWS + j * K
            wvs = [
                plsc.load_gather(
                    w_v, [jnp.full((LANES,), e0 + k, dtype=jnp.int32)])
                for k in range(K)
            ]
            bv = plsc.load_gather(
                bias_v, [jnp.full((LANES,), c * CH + j, dtype=jnp.int32)])

            @plsc.parallel_loop(0, TB, unroll=2)
            def _tile(t):
                a = bv
                for k in range(K):
                    a = a + wvs[k] * rows_v[j * K + k,
                                            pl.ds(t * LANES, LANES)]
                if apply_act:
                    a = a / (1.0 + jnp.exp(-a))
                acc_v[j, pl.ds(t * LANES, LANES)] = a

    ghandles = {0: gather(0)}
    shandles = []
    for c in range(NCHUNK):
        ghandles[c].wait()
        if c + 1 < NCHUNK:
            ghandles[c + 1] = gather(c + 1)
        if c >= 2:
            shandles[c - 2].wait()
        compute(c)
        shandles.append(pltpu.async_copy(
            acc[c % 2], out_hbm.at[pl.ds(base + c * CH, CH)], ssem))
    for h in shandles[-2:]:
        h.wait()


def _make_layer(apply_act, interpret=False):
    return pl.kernel(
        functools.partial(_layer_body, apply_act),
        out_type=jax.ShapeDtypeStruct((N, B), jnp.float32),
        mesh=plsc.VectorSubcoreMesh(
            core_axis_name="c", subcore_axis_name="s",
            num_cores=NC, num_subcores=NS),
        scratch_types=[
            pltpu.VMEM((NPW * K,), jnp.int32),
            pltpu.VMEM((NPW * K,), jnp.float32),
            pltpu.VMEM((NPW,), jnp.float32),
            pltpu.VMEM((ROWS, B), jnp.float32),
            pltpu.VMEM((ROWS, B), jnp.float32),
            pltpu.VMEM((CH, B), jnp.float32),
            pltpu.VMEM((CH, B), jnp.float32),
            pltpu.SemaphoreType.DMA,
            pltpu.SemaphoreType.DMA,
            pltpu.SemaphoreType.DMA,
        ],
        compiler_params=pltpu.CompilerParams(needs_layout_passes=False),
        interpret=interpret,
    )


@jax.jit
def kernel(x, edge_src_local, edge_w, bias):
    vals = x.T
    hidden = _make_layer(True)
    final = _make_layer(False)
    for l in range(NL - 1):
        fn = hidden if l < NL - 2 else final
        vals = fn(vals, edge_src_local[l].reshape(-1),
                  edge_w[l].reshape(-1), bias[l])
    return vals.T


# speedup vs baseline: 5.0617x; 1.0069x over previous
import functools

import jax
import jax.numpy as jnp
from jax import lax
from jax.experimental import pallas as pl
from jax.experimental.pallas import tpu as pltpu
from jax.experimental.pallas import tpu_sc as plsc

NL = 8
N = 4096
K = 16
B = 256
NC, NS, LANES = 2, 16, 16
NW = NC * NS
NPW = N // NW
CH = 8
NCHUNK = NPW // CH
ROWS = CH * K
TB = B // LANES


def _layer_body(apply_act, vals_hbm, src_hbm, w_hbm, bias_hbm, out_hbm,
                idx_v, w_v, bias_v, rows0, rows1, acc0, acc1,
                gsem0, gsem1, ssem):
    wid = lax.axis_index("s") * NC + lax.axis_index("c")
    base = wid * NPW
    ebase = base * K

    rows = (rows0, rows1)
    acc = (acc0, acc1)
    gsem = (gsem0, gsem1)

    def gather(c):
        return pltpu.async_copy(
            vals_hbm.at[idx_v.at[pl.ds(c * ROWS, ROWS)]], rows[c % 2],
            gsem[c % 2])

    pltpu.sync_copy(src_hbm.at[pl.ds(ebase, NPW * K)], idx_v)
    first = gather(0)
    pltpu.sync_copy(w_hbm.at[pl.ds(ebase, NPW * K)], w_v)
    pltpu.sync_copy(bias_hbm.at[pl.ds(base, NPW)], bias_v)

    def compute(c):
        rows_v, acc_v = rows[c % 2], acc[c % 2]

        @plsc.parallel_loop(0, CH)
        def _neuron(j):
            e0 = c * ROWS + j * K
            wvs = [
                plsc.load_gather(
                    w_v, [jnp.full((LANES,), e0 + k, dtype=jnp.int32)])
                for k in range(K)
            ]
            bv = plsc.load_gather(
                bias_v, [jnp.full((LANES,), c * CH + j, dtype=jnp.int32)])

            @plsc.parallel_loop(0, TB, unroll=4)
            def _tile(t):
                terms = [wvs[k] * rows_v[j * K + k, pl.ds(t * LANES, LANES)]
                         for k in range(K)]
                while len(terms) > 1:
                    terms = [terms[i] + terms[i + 1]
                             for i in range(0, len(terms) - 1, 2)] + \
                            (terms[-1:] if len(terms) % 2 else [])
                a = terms[0] + bv
                if apply_act:
                    a = a / (1.0 + jnp.exp(-a))
                acc_v[j, pl.ds(t * LANES, LANES)] = a

    ghandles = {0: first}
    shandles = []
    for c in range(NCHUNK):
        ghandles[c].wait()
        if c + 1 < NCHUNK:
            ghandles[c + 1] = gather(c + 1)
        if c >= 2:
            shandles[c - 2].wait()
        compute(c)
        shandles.append(pltpu.async_copy(
            acc[c % 2], out_hbm.at[pl.ds(base + c * CH, CH)], ssem))
    for h in shandles[-2:]:
        h.wait()


def _make_layer(apply_act, interpret=False):
    return pl.kernel(
        functools.partial(_layer_body, apply_act),
        out_type=jax.ShapeDtypeStruct((N, B), jnp.float32),
        mesh=plsc.VectorSubcoreMesh(
            core_axis_name="c", subcore_axis_name="s",
            num_cores=NC, num_subcores=NS),
        scratch_types=[
            pltpu.VMEM((NPW * K,), jnp.int32),
            pltpu.VMEM((NPW * K,), jnp.float32),
            pltpu.VMEM((NPW,), jnp.float32),
            pltpu.VMEM((ROWS, B), jnp.float32),
            pltpu.VMEM((ROWS, B), jnp.float32),
            pltpu.VMEM((CH, B), jnp.float32),
            pltpu.VMEM((CH, B), jnp.float32),
            pltpu.SemaphoreType.DMA,
            pltpu.SemaphoreType.DMA,
            pltpu.SemaphoreType.DMA,
        ],
        compiler_params=pltpu.CompilerParams(needs_layout_passes=False),
        interpret=interpret,
    )


@jax.jit
def kernel(x, edge_src_local, edge_w, bias):
    vals = x.T
    hidden = _make_layer(True)
    final = _make_layer(False)
    for l in range(NL - 1):
        fn = hidden if l < NL - 2 else final
        vals = fn(vals, edge_src_local[l].reshape(-1),
                  edge_w[l].reshape(-1), bias[l])
    return vals.T


# speedup vs baseline: 5.6422x; 1.1147x over previous
import functools

import jax
import jax.numpy as jnp
from jax import lax
from jax.experimental import pallas as pl
from jax.experimental.pallas import tpu as pltpu
from jax.experimental.pallas import tpu_sc as plsc

NL = 8
N = 4096
K = 16
B = 256
NC, NS, LANES = 2, 16, 16
NW = NC * NS
NPW = N // NW
CH = 8
NCHUNK = NPW // CH
ROWS = CH * K
W2 = B // 2
GW = W2 // LANES


def _tree_sum(terms):
    while len(terms) > 1:
        terms = [terms[i] + terms[i + 1]
                 for i in range(0, len(terms) - 1, 2)] + \
                (terms[-1:] if len(terms) % 2 else [])
    return terms[0]


def _layer_body(apply_act, f32_out, vals_hbm, src_hbm, w_hbm, bias_hbm,
                out_hbm, idx_v, w_v, bias_v, rows0, rows1, acc0, acc1,
                gsem0, gsem1, ssem):
    wid = lax.axis_index("s") * NC + lax.axis_index("c")
    base = wid * NPW
    ebase = base * K

    rows = (rows0, rows1)
    acc = (acc0, acc1)
    gsem = (gsem0, gsem1)

    def gather(c):
        return pltpu.async_copy(
            vals_hbm.at[idx_v.at[pl.ds(c * ROWS, ROWS)]], rows[c % 2],
            gsem[c % 2])

    pltpu.sync_copy(src_hbm.at[pl.ds(ebase, NPW * K)], idx_v)
    first = gather(0)
    pltpu.sync_copy(w_hbm.at[pl.ds(ebase, NPW * K)], w_v)
    pltpu.sync_copy(bias_hbm.at[pl.ds(base, NPW)], bias_v)

    def compute(c):
        rows_v, acc_v = rows[c % 2], acc[c % 2]

        @plsc.parallel_loop(0, CH)
        def _neuron(j):
            e0 = c * ROWS + j * K
            wvs = [
                plsc.load_gather(
                    w_v, [jnp.full((LANES,), e0 + k, dtype=jnp.int32)])
                for k in range(K)
            ]
            bv = plsc.load_gather(
                bias_v, [jnp.full((LANES,), c * CH + j, dtype=jnp.int32)])

            @plsc.parallel_loop(0, GW, unroll=2)
            def _grp(g):
                ta, tb = [], []
                for k in range(K):
                    words = rows_v[j * K + k, pl.ds(g * LANES, LANES)]
                    ra, rb = plsc.unpack(
                        plsc.bitcast(words, jnp.bfloat16),
                        format=plsc.PackFormat.INTERLEAVED)
                    ta.append(wvs[k] * ra)
                    tb.append(wvs[k] * rb)
                a = _tree_sum(ta) + bv
                b = _tree_sum(tb) + bv
                if apply_act:
                    a = a / (1.0 + jnp.exp(-a))
                    b = b / (1.0 + jnp.exp(-b))
                if f32_out:
                    row_i = jnp.full((LANES,), j, dtype=jnp.int32)
                    cols = 32 * g + 2 * lax.iota(jnp.int32, LANES)
                    plsc.store_scatter(acc_v, [row_i, cols], a)
                    plsc.store_scatter(acc_v, [row_i, cols + 1], b)
                else:
                    acc_v[j, pl.ds(g * LANES, LANES)] = plsc.bitcast(
                        plsc.pack(a, b, format=plsc.PackFormat.INTERLEAVED),
                        jnp.int32)

    ghandles = {0: first}
    shandles = []
    for c in range(NCHUNK):
        ghandles[c].wait()
        if c + 1 < NCHUNK:
            ghandles[c + 1] = gather(c + 1)
        if c >= 2:
            shandles[c - 2].wait()
        compute(c)
        shandles.append(pltpu.async_copy(
            acc[c % 2], out_hbm.at[pl.ds(base + c * CH, CH)], ssem))
    for h in shandles[-2:]:
        h.wait()


def _make_layer(apply_act, f32_out):
    out_w = B if f32_out else W2
    out_t = jnp.float32 if f32_out else jnp.int32
    return pl.kernel(
        functools.partial(_layer_body, apply_act, f32_out),
        out_type=jax.ShapeDtypeStruct((N, out_w), out_t),
        mesh=plsc.VectorSubcoreMesh(
            core_axis_name="c", subcore_axis_name="s",
            num_cores=NC, num_subcores=NS),
        scratch_types=[
            pltpu.VMEM((NPW * K,), jnp.int32),
            pltpu.VMEM((NPW * K,), jnp.float32),
            pltpu.VMEM((NPW,), jnp.float32),
            pltpu.VMEM((ROWS, W2), jnp.int32),
            pltpu.VMEM((ROWS, W2), jnp.int32),
            pltpu.VMEM((CH, out_w), out_t),
            pltpu.VMEM((CH, out_w), out_t),
            pltpu.SemaphoreType.DMA,
            pltpu.SemaphoreType.DMA,
            pltpu.SemaphoreType.DMA,
        ],
        compiler_params=pltpu.CompilerParams(needs_layout_passes=False),
    )


@jax.jit
def kernel(x, edge_src_local, edge_w, bias):
    xt = x.T.astype(jnp.bfloat16)
    vals = lax.bitcast_convert_type(
        xt.reshape(N, W2, 2), jnp.int32)
    hidden = _make_layer(True, False)
    final = _make_layer(False, True)
    for l in range(NL - 1):
        fn = hidden if l < NL - 2 else final
        vals = fn(vals, edge_src_local[l].reshape(-1),
                  edge_w[l].reshape(-1), bias[l])
    return vals.T
